# baseline (speedup 1.0000x reference)
import jax
import jax.numpy as jnp
from jax import lax
from jax.experimental import pallas as pl
from jax.experimental.pallas import tpu as pltpu
from jax.experimental.pallas import tpu_sc as plsc

_B, _T, _V = 160, 16, 10000
_TAU = 0.13
_ALPHA = 0.7

_NC, _NS, _L = 2, 16, 16
_NW = _NC * _NS
_N = _B * _T
_TPW = _N // _NW

_KG = 4
_NST = _TPW // _KG

_R = 128
_NSTEP = _N // _R


def _sc_stage_body(sim, tgt_pad, out, idx2_v, rows_v,
                   sem_g0, sem_g1, sem_s0, sem_s1):
    wid = lax.axis_index("s") * _NC + lax.axis_index("c")
    base = wid * _TPW
    pltpu.sync_copy(tgt_pad.at[pl.ds(wid * _NST, _NST)], idx2_v)
    sem_g = (sem_g0, sem_g1)
    sem_s = (sem_s0, sem_s1)

    def gather(st, b):
        return pltpu.async_copy(sim.at[idx2_v.at[st, pl.ds(0, _KG)]],
                                rows_v.at[b], sem_g[b])

    def scatter(st, b):
        return pltpu.async_copy(rows_v.at[b],
                                out.at[pl.ds(base + st * _KG, _KG)], sem_s[b])

    hg = {0: gather(0, 0)}
    hs = {}
    for st in range(_NST):
        b = st % 2
        hg.pop(st).wait()
        if st + 1 < _NST:
            if st >= 1:
                hs.pop(st - 1).wait()
            hg[st + 1] = gather(st + 1, (st + 1) % 2)
        hs[st] = scatter(st, b)
    for st in sorted(hs):
        hs.pop(st).wait()


@jax.jit
def _sc_stage(sim, tgt_pad):
    mesh = plsc.VectorSubcoreMesh(core_axis_name="c", subcore_axis_name="s",
                                  num_cores=_NC, num_subcores=_NS)
    f = pl.kernel(
        _sc_stage_body,
        out_type=jax.ShapeDtypeStruct((_N, _V), jnp.float32),
        mesh=mesh,
        compiler_params=pltpu.CompilerParams(needs_layout_passes=False,
                                             use_tc_tiling_on_sc=False),
        scratch_types=[
            pltpu.VMEM((_NST, 8), jnp.int32),
            pltpu.VMEM((2, _KG, _V), jnp.float32),
            pltpu.SemaphoreType.DMA,
            pltpu.SemaphoreType.DMA,
            pltpu.SemaphoreType.DMA,
            pltpu.SemaphoreType.DMA,
        ],
    )
    return f(sim, tgt_pad)


def _sc_nll_body(inpflat, tgt, maskv, out,
                 idx_v, flatidx_v, mask_v, mlvals_v, stage_v, sem_ml, sem_out):
    wid = lax.axis_index("s") * _NC + lax.axis_index("c")
    base = wid * _TPW
    iota = lax.broadcasted_iota(jnp.int32, (_L,), 0)

    pltpu.sync_copy(tgt.at[pl.ds(base, _TPW)], idx_v)
    pltpu.sync_copy(maskv.at[pl.ds(base, _TPW)], mask_v)

    for k in range(_TPW // _L):
        sl = pl.ds(k * _L, _L)
        rowid = iota + (base + k * _L)
        flatidx_v[sl] = rowid * _V + idx_v[sl]
    pltpu.async_copy(inpflat.at[flatidx_v], mlvals_v, sem_ml).wait()

    zeros = jnp.zeros((_L,), jnp.float32)
    mlacc = zeros
    msacc = zeros
    for k in range(_TPW // _L):
        sl = pl.ds(k * _L, _L)
        m16 = mask_v[sl]
        mlacc = mlacc + mlvals_v[sl] * m16
        msacc = msacc + m16
    stage = jnp.where(iota == 0, jnp.sum(mlacc), 0.0)
    stage = stage + jnp.where(iota == 1, jnp.sum(msacc), 0.0)
    stage_v[...] = stage
    pltpu.async_copy(stage_v, out.at[wid], sem_out).wait()


@jax.jit
def _sc_nll(inpflat, tgt, maskv):
    mesh = plsc.VectorSubcoreMesh(core_axis_name="c", subcore_axis_name="s",
                                  num_cores=_NC, num_subcores=_NS)
    f = pl.kernel(
        _sc_nll_body,
        out_type=jax.ShapeDtypeStruct((_NW, _L), jnp.float32),
        mesh=mesh,
        compiler_params=pltpu.CompilerParams(needs_layout_passes=False,
                                             use_tc_tiling_on_sc=False),
        scratch_types=[
            pltpu.VMEM((_TPW,), jnp.int32),
            pltpu.VMEM((_TPW,), jnp.int32),
            pltpu.VMEM((_TPW,), jnp.float32),
            pltpu.VMEM((_TPW,), jnp.float32),
            pltpu.VMEM((_L,), jnp.float32),
            pltpu.SemaphoreType.DMA,
            pltpu.SemaphoreType.DMA,
        ],
    )
    return f(inpflat, tgt, maskv)


def _tc_body(staged_ref, inp_ref, mask_ref, o_sm_ref):
    i = pl.program_id(0)
    inv_tau = jnp.float32(1.0 / _TAU)
    e = jnp.exp(staged_ref[...] * inv_tau)
    s8 = jnp.sum(e, axis=1, keepdims=True)
    d8 = jnp.sum(e * inp_ref[...], axis=1, keepdims=True)
    contrib = jnp.sum(mask_ref[...] * d8 / s8)

    @pl.when(i == 0)
    def _():
        o_sm_ref[0, 0] = 0.0

    o_sm_ref[0, 0] += contrib


@jax.jit
def _tc_smooth(staged, inp2, mask2d):
    return pl.pallas_call(
        _tc_body,
        grid=(_NSTEP,),
        in_specs=[
            pl.BlockSpec((_R, _V), lambda i: (i, 0)),
            pl.BlockSpec((_R, _V), lambda i: (i, 0)),
            pl.BlockSpec((_R, 1), lambda i: (i, 0)),
        ],
        out_specs=pl.BlockSpec((1, 1), lambda i: (0, 0),
                               memory_space=pltpu.MemorySpace.SMEM),
        out_shape=jax.ShapeDtypeStruct((1, 1), jnp.float32),
        compiler_params=pltpu.CompilerParams(
            dimension_semantics=("arbitrary",)),
    )(staged, inp2, mask2d)


def kernel(input, target, mask, sim_matrix):
    inp2 = input.reshape(_N, _V)
    inpflat = input.reshape(_N * _V)
    tgt = target.reshape(_N).astype(jnp.int32)
    tgt_pad = jnp.pad(tgt.reshape(_N // _KG, _KG), ((0, 0), (0, 8 - _KG)))
    maskv = mask.reshape(_N)
    mask2d = mask.reshape(_N, 1)

    staged = _sc_stage(sim_matrix, tgt_pad)
    parts = _sc_nll(inpflat, tgt, maskv)
    smooth_sum = _tc_smooth(staged, inp2, mask2d)[0, 0]

    ml_sum = jnp.sum(parts[:, 0])
    msum = jnp.sum(parts[:, 1])
    ml_output = -ml_sum / msum
    output = _ALPHA * (-smooth_sum / msum) + (1.0 - _ALPHA) * ml_output
    return (ml_output, output)
